# baseline (speedup 1.0000x reference)
import functools

import jax
import jax.numpy as jnp
from jax import lax
from jax.experimental import pallas as pl
from jax.experimental.pallas import tpu as pltpu, tpu_sc as plsc

NUM_EMBEDDINGS = 100000

NC = 2
NS = 16
NW = NC * NS
LANES = 16
CHUNK = 128
GRP = 8
SUBV = CHUNK // LANES


def kernel(x, table):
    B, F = x.shape
    D = table.shape[-1]
    total = B * F
    per_w = total // NW
    n_chunks = per_w // CHUNK
    n_grp = n_chunks // GRP
    assert per_w * NW == total and n_grp * GRP == n_chunks
    assert per_w % F == 0

    rows_g = GRP * CHUNK
    x_r = x.reshape(NW, n_grp, rows_g)
    mesh = plsc.VectorSubcoreMesh(core_axis_name="c", subcore_axis_name="s")

    @functools.partial(
        pl.kernel,
        mesh=mesh,
        compiler_params=pltpu.CompilerParams(use_tc_tiling_on_sc=False),
        out_type=jax.ShapeDtypeStruct((NW, n_grp, rows_g, D), jnp.float32),
        scratch_types=[
            pltpu.VMEM((n_grp, rows_g), jnp.int32),
            pltpu.VMEM((2, rows_g, D), jnp.float32),
            pltpu.SemaphoreType.DMA((2,)),
            pltpu.SemaphoreType.DMA((2,)),
        ],
    )
    def k(x_hbm, tab_hbm, out_hbm, idx_v, rows_v, gsem, ssem):
        wid = lax.axis_index("s") * NC + lax.axis_index("c")
        pltpu.sync_copy(x_hbm.at[wid], idx_v)

        lane = lax.iota(jnp.int32, LANES)
        wrap = jnp.int32(F)

        def adj_body(g, f_vec):
            for i in range(GRP * SUBV):
                sl = pl.ds(i * LANES, LANES)
                idx_v[g, sl] = idx_v[g, sl] + f_vec * NUM_EMBEDDINGS
                t = f_vec + LANES
                f_vec = lax.select(t >= wrap, t - wrap, t)
            return f_vec

        lax.fori_loop(0, n_grp, adj_body, lane)

        U = LANES

        def fire_gather(g, b):
            def issue(i, _):
                v = idx_v[g, pl.ds(i * U, U)]
                for u in range(U):
                    pltpu.async_copy(
                        tab_hbm.at[pl.ds(v[u], 1)],
                        rows_v.at[b, pl.ds(i * U + u, 1)], gsem.at[b])
                return ()

            lax.fori_loop(0, rows_g // U, issue, ())

        def wait_gather(g, b):
            pltpu.make_async_copy(
                tab_hbm.at[pl.ds(0, rows_g)],
                rows_v.at[b], gsem.at[b]).wait()

        def fire_scatter(g, b):
            pltpu.async_copy(rows_v.at[b], out_hbm.at[wid, g], ssem.at[b])

        def wait_scatter(g, b):
            pltpu.make_async_copy(
                rows_v.at[b], out_hbm.at[wid, g], ssem.at[b]).wait()

        fire_gather(0, 0)

        def body(g, _):
            b = lax.rem(g, 2)

            @pl.when(g + 1 < n_grp)
            def _():
                @pl.when(g >= 1)
                def _():
                    wait_scatter(g - 1, 1 - b)
                fire_gather(g + 1, 1 - b)

            wait_gather(g, b)
            fire_scatter(g, b)
            return ()

        lax.fori_loop(0, n_grp, body, ())
        wait_scatter(n_grp - 2, (n_grp - 2) % 2)
        wait_scatter(n_grp - 1, (n_grp - 1) % 2)

    out = k(x_r, table)
    return out.reshape(B, F, D)


# speedup vs baseline: 1.0055x; 1.0055x over previous
import functools

import jax
import jax.numpy as jnp
from jax import lax
from jax.experimental import pallas as pl
from jax.experimental.pallas import tpu as pltpu, tpu_sc as plsc

NUM_EMBEDDINGS = 100000

NC = 2
NS = 16
NW = NC * NS
LANES = 16
CHUNK = 128
GRP = 8
SUBV = CHUNK // LANES


def kernel(x, table):
    B, F = x.shape
    D = table.shape[-1]
    total = B * F
    per_w = total // NW
    n_chunks = per_w // CHUNK
    n_grp = n_chunks // GRP
    assert per_w * NW == total and n_grp * GRP == n_chunks
    assert per_w % F == 0

    rows_g = GRP * CHUNK
    x_r = x.reshape(NW, n_grp, rows_g)
    mesh = plsc.VectorSubcoreMesh(core_axis_name="c", subcore_axis_name="s")

    @functools.partial(
        pl.kernel,
        mesh=mesh,
        compiler_params=pltpu.CompilerParams(use_tc_tiling_on_sc=False),
        out_type=jax.ShapeDtypeStruct((NW, n_grp, rows_g, D), jnp.float32),
        scratch_types=[
            pltpu.VMEM((n_grp, rows_g), jnp.int32),
            pltpu.VMEM((2, rows_g, D), jnp.float32),
            pltpu.SemaphoreType.DMA((2,)),
            pltpu.SemaphoreType.DMA((2,)),
        ],
    )
    def k(x_hbm, tab_hbm, out_hbm, idx_v, rows_v, gsem, ssem):
        wid = lax.axis_index("s") * NC + lax.axis_index("c")
        pltpu.sync_copy(x_hbm.at[wid], idx_v)

        lane = lax.iota(jnp.int32, LANES)
        wrap = jnp.int32(F)

        def adj_body(g, f_vec):
            for i in range(GRP * SUBV):
                sl = pl.ds(i * LANES, LANES)
                idx_v[g, sl] = idx_v[g, sl] + f_vec * NUM_EMBEDDINGS
                t = f_vec + LANES
                f_vec = lax.select(t >= wrap, t - wrap, t)
            return f_vec

        lax.fori_loop(0, n_grp, adj_body, lane)

        U = LANES

        def fire_gather(g, b):
            @pl.when(lax.rem(g, 2) == 0)
            def _():
                pltpu.async_copy(
                    tab_hbm.at[idx_v.at[g]], rows_v.at[b], gsem.at[b])

            @pl.when(lax.rem(g, 2) == 1)
            def _():
                def issue(i, _):
                    v = idx_v[g, pl.ds(i * U, U)]
                    for u in range(U):
                        pltpu.async_copy(
                            tab_hbm.at[pl.ds(v[u], 1)],
                            rows_v.at[b, pl.ds(i * U + u, 1)], gsem.at[b])
                    return ()

                lax.fori_loop(0, rows_g // U, issue, ())

        def wait_gather(g, b):
            pltpu.make_async_copy(
                tab_hbm.at[pl.ds(0, rows_g)],
                rows_v.at[b], gsem.at[b]).wait()

        def fire_scatter(g, b):
            pltpu.async_copy(rows_v.at[b], out_hbm.at[wid, g], ssem.at[b])

        def wait_scatter(g, b):
            pltpu.make_async_copy(
                rows_v.at[b], out_hbm.at[wid, g], ssem.at[b]).wait()

        fire_gather(0, 0)

        def body(g, _):
            b = lax.rem(g, 2)

            @pl.when(g + 1 < n_grp)
            def _():
                @pl.when(g >= 1)
                def _():
                    wait_scatter(g - 1, 1 - b)
                fire_gather(g + 1, 1 - b)

            wait_gather(g, b)
            fire_scatter(g, b)
            return ()

        lax.fori_loop(0, n_grp, body, ())
        wait_scatter(n_grp - 2, (n_grp - 2) % 2)
        wait_scatter(n_grp - 1, (n_grp - 1) % 2)

    out = k(x_r, table)
    return out.reshape(B, F, D)
